# baseline (speedup 1.0000x reference)
import functools

import jax
import jax.numpy as jnp
from jax import lax
from jax.experimental import pallas as pl
from jax.experimental.pallas import tpu as pltpu
from jax.experimental.pallas import tpu_sc as plsc

_TILE_V = 8192
_NEG = -3e38
_LOG2E = 1.4426950408889634
_LN2 = 0.6931471805599453


@functools.lru_cache(maxsize=None)
def _make_sc_gather(B, Wn, V, D):
    info = plsc.get_sparse_core_info()
    NC, NS = info.num_cores, info.num_subcores
    NW = NC * NS
    assert D % NW == 0
    d_per_w = D // NW
    assert B % 16 == 0 and (B * Wn) % 128 == 0

    mesh = plsc.VectorSubcoreMesh(core_axis_name="c", subcore_axis_name="s")

    scratch = [
        pltpu.VMEM((B,), jnp.int32),
        pltpu.VMEM((B * Wn,), jnp.int32),
        pltpu.VMEM((1, V), jnp.float32),
        pltpu.VMEM((1, B), jnp.float32),
        pltpu.VMEM((1, B), jnp.float32),
        pltpu.SemaphoreType.DMA,
    ]

    @functools.partial(
        pl.kernel,
        mesh=mesh,
        out_type=(jax.ShapeDtypeStruct((D, B), jnp.float32),
                  jax.ShapeDtypeStruct((D, B), jnp.float32)),
        scratch_types=scratch,
        compiler_params=pltpu.CompilerParams(use_tc_tiling_on_sc=False,
                                             needs_layout_passes=False),
    )
    def sc_gather(cidx_hbm, ctx_hbm, wembt_hbm, woutt_hbm, out_ht, out_csumt,
                  cidx_v, ctx_v, row_v, hrow, crow, sem):
        wid = lax.axis_index("s") * NC + lax.axis_index("c")

        pltpu.sync_copy(cidx_hbm, cidx_v)
        pltpu.sync_copy(ctx_hbm, ctx_v)

        lane = lax.iota(jnp.int32, 16)
        zero16 = jnp.zeros((16,), jnp.int32)
        posw = lane * Wn

        def center_g(g, carry):
            idxv = cidx_v[pl.ds(g * 16, 16)]
            hrow[0, pl.ds(g * 16, 16)] = plsc.load_gather(row_v, [zero16, idxv])
            return carry

        def ctx_g(g, carry):
            acc = None
            for w in range(Wn):
                posv = posw + (g * 128 + w)
                cv = plsc.load_gather(ctx_v, [posv])
                vals = plsc.load_gather(row_v, [zero16, cv])
                acc = vals if acc is None else acc + vals
            crow[0, pl.ds(g * 16, 16)] = acc
            return carry

        for d_off in range(d_per_w):
            d = wid * d_per_w + d_off
            pltpu.sync_copy(wembt_hbm.at[pl.ds(d, 1), :], row_v)
            lax.fori_loop(0, B // 16, center_g, 0)
            pltpu.sync_copy(hrow, out_ht.at[pl.ds(d, 1), :])

            pltpu.sync_copy(woutt_hbm.at[pl.ds(d, 1), :], row_v)
            lax.fori_loop(0, B // 16, ctx_g, 0)
            pltpu.sync_copy(crow, out_csumt.at[pl.ds(d, 1), :])

    return sc_gather


def _tc_body(ht_ref, csumt_ref, woutt_ref, out_ref, m_ref, s_ref, hb_ref,
             hn_ref, *, B, Wn, V, tile_v, n_pad):
    i = pl.program_id(0)
    nt = pl.num_programs(0)

    @pl.when(i == 0)
    def _init():
        m_ref[...] = jnp.full(m_ref.shape, _NEG, jnp.float32)
        s_ref[...] = jnp.zeros(s_ref.shape, jnp.float32)
        h = jnp.transpose(ht_ref[...], (1, 0))
        hb_ref[...] = (h * _LOG2E).astype(jnp.bfloat16)
        hn_ref[...] = jnp.sqrt(jnp.sum(h * h, axis=1, keepdims=True)) * _LOG2E

    wt = woutt_ref[...]
    col = i * tile_v + lax.broadcasted_iota(jnp.int32, (1, tile_v), 1)
    wt = jnp.where(col < V, wt, 0.0)
    rn2 = jnp.sum(wt * wt, axis=0, keepdims=True)
    mi = jnp.sqrt(jnp.max(rn2, axis=1, keepdims=True))
    b2w = (hn_ref[...] * mi).astype(jnp.bfloat16)
    b2 = b2w.astype(jnp.float32)

    hb = jnp.concatenate([hb_ref[...], b2w], axis=1)
    wb = jnp.concatenate(
        [wt.astype(jnp.bfloat16),
         jnp.full((1, tile_v), -1.0, jnp.bfloat16)], axis=0)
    l2mb = lax.dot_general(hb, wb, (((1,), (0,)), ((), ())),
                           preferred_element_type=jnp.float32)
    t = jnp.sum(jnp.exp2(l2mb), axis=1, keepdims=True)

    m_old = m_ref[...]
    m_new = jnp.maximum(m_old, b2)
    s_ref[...] = (s_ref[...] * jnp.exp2(m_old - m_new)
                  + t * jnp.exp2(b2 - m_new))
    m_ref[...] = m_new

    @pl.when(i == nt - 1)
    def _fin():
        m = m_ref[...]
        s = s_ref[...] - n_pad * jnp.exp2(-m)
        lse = m * _LN2 + jnp.log(s)
        ctx_total = jnp.sum(ht_ref[...] * csumt_ref[...])
        out_ref[0, 0] = jnp.sum(lse) / B - ctx_total / (B * Wn)


@functools.lru_cache(maxsize=None)
def _make_tc_loss(B, Wn, V, D):
    tile_v = _TILE_V
    nt = pl.cdiv(V, tile_v)
    n_pad = nt * tile_v - V
    body = functools.partial(_tc_body, B=B, Wn=Wn, V=V, tile_v=tile_v,
                             n_pad=n_pad)
    return pl.pallas_call(
        body,
        grid=(nt,),
        in_specs=[
            pl.BlockSpec((D, B), lambda i: (0, 0)),
            pl.BlockSpec((D, B), lambda i: (0, 0)),
            pl.BlockSpec((D, tile_v), lambda i: (0, i)),
        ],
        out_specs=pl.BlockSpec((1, 1), lambda i: (0, 0),
                               memory_space=pltpu.SMEM),
        out_shape=jax.ShapeDtypeStruct((1, 1), jnp.float32),
        scratch_shapes=[
            pltpu.VMEM((B, 1), jnp.float32),
            pltpu.VMEM((B, 1), jnp.float32),
            pltpu.VMEM((B, D), jnp.bfloat16),
            pltpu.VMEM((B, 1), jnp.float32),
        ],
    )


def kernel(center_index, context_indices, W_emb, W_out):
    B, Wn = context_indices.shape
    V, D = W_emb.shape
    cidx = center_index.astype(jnp.int32)
    ctx = context_indices.astype(jnp.int32).reshape(-1)
    wembt = jnp.swapaxes(W_emb, 0, 1)
    woutt = jnp.swapaxes(W_out, 0, 1)
    ht, csumt = _make_sc_gather(B, Wn, V, D)(cidx, ctx, wembt, woutt)
    loss = _make_tc_loss(B, Wn, V, D)(ht, csumt, woutt)
    return loss[0, 0]


# speedup vs baseline: 1.1517x; 1.1517x over previous
import functools

import jax
import jax.numpy as jnp
from jax import lax
from jax.experimental import pallas as pl
from jax.experimental.pallas import tpu as pltpu
from jax.experimental.pallas import tpu_sc as plsc

_TILE_V = 8192
_NEG = -3e38
_LOG2E = 1.4426950408889634
_LN2 = 0.6931471805599453


@functools.lru_cache(maxsize=None)
def _make_sc_ctx(B, Wn, V, D):
    info = plsc.get_sparse_core_info()
    NC, NS = info.num_cores, info.num_subcores
    NW = NC * NS
    assert D % NW == 0
    d_per_w = D // NW
    assert B % 16 == 0 and (B * Wn) % 128 == 0

    mesh = plsc.VectorSubcoreMesh(core_axis_name="c", subcore_axis_name="s")

    scratch = [
        pltpu.VMEM((B * Wn,), jnp.int32),
        pltpu.VMEM((1, V), jnp.float32),
        pltpu.VMEM((1, B), jnp.float32),
    ]

    @functools.partial(
        pl.kernel,
        mesh=mesh,
        out_type=jax.ShapeDtypeStruct((D, B), jnp.float32),
        scratch_types=scratch,
        compiler_params=pltpu.CompilerParams(use_tc_tiling_on_sc=False,
                                             needs_layout_passes=False),
    )
    def sc_ctx(ctx_hbm, woutt_hbm, out_csumt, ctx_v, row_v, crow):
        wid = lax.axis_index("s") * NC + lax.axis_index("c")

        pltpu.sync_copy(ctx_hbm, ctx_v)

        lane = lax.iota(jnp.int32, 16)
        zero16 = jnp.zeros((16,), jnp.int32)
        posw = lane * Wn

        def ctx_g(g, carry):
            acc = None
            for w in range(Wn):
                posv = posw + (g * 128 + w)
                cv = plsc.load_gather(ctx_v, [posv])
                vals = plsc.load_gather(row_v, [zero16, cv])
                acc = vals if acc is None else acc + vals
            crow[0, pl.ds(g * 16, 16)] = acc
            return carry

        for d_off in range(d_per_w):
            d = wid * d_per_w + d_off
            pltpu.sync_copy(woutt_hbm.at[pl.ds(d, 1), :], row_v)
            lax.fori_loop(0, B // 16, ctx_g, 0)
            pltpu.sync_copy(crow, out_csumt.at[pl.ds(d, 1), :])

    return sc_ctx


@functools.lru_cache(maxsize=None)
def _make_sc_center(B, V, D):
    info = plsc.get_sparse_core_info()
    NC, NS = info.num_cores, info.num_subcores
    NW = NC * NS
    assert B % NW == 0
    b_per_w = B // NW
    assert b_per_w % 16 == 0 and D % 16 == 0
    nl = D // 16

    mesh = plsc.VectorSubcoreMesh(core_axis_name="c", subcore_axis_name="s")

    scratch = [
        pltpu.VMEM((b_per_w,), jnp.int32),
        pltpu.VMEM((D, 128), jnp.float32),
        pltpu.VMEM((D, 128), jnp.float32),
        pltpu.VMEM((b_per_w, D), jnp.float32),
        pltpu.SemaphoreType.DMA,
        pltpu.SemaphoreType.DMA,
    ]

    @functools.partial(
        pl.kernel,
        mesh=mesh,
        out_type=jax.ShapeDtypeStruct((B, D), jnp.float32),
        scratch_types=scratch,
        compiler_params=pltpu.CompilerParams(needs_layout_passes=False),
    )
    def sc_center(cidx_hbm, wembt_hbm, out_h, idx_v, blk_a, blk_b, hstage,
                  sem_a, sem_b):
        wid = lax.axis_index("s") * NC + lax.axis_index("c")
        base = wid * b_per_w

        pltpu.sync_copy(cidx_hbm.at[pl.ds(base, b_per_w)], idx_v)

        lane = lax.iota(jnp.int32, 16)
        bufs = (blk_a, blk_b)
        sems = (sem_a, sem_b)

        def fetch(v, buf, sem):
            blk0 = pl.multiple_of((v // 128) * 128, 128)
            return pltpu.async_copy(wembt_hbm.at[:, pl.ds(blk0, 128)], buf,
                                    sem)

        def extract(v, buf, j):
            off = jnp.full((16,), v - (v // 128) * 128, jnp.int32)
            for k in range(nl):
                vals = plsc.load_gather(buf, [lane + (16 * k), off])
                hstage[j, pl.ds(16 * k, 16)] = vals

        for j0 in range(0, b_per_w, 16):
            vvec = idx_v[pl.ds(j0, 16)]
            cp = fetch(vvec[0], bufs[0], sems[0])
            for j in range(16):
                nxt = None
                if j + 1 < 16:
                    nxt = fetch(vvec[j + 1], bufs[(j + 1) % 2], sems[(j + 1) % 2])
                cp.wait()
                extract(vvec[j], bufs[j % 2], j0 + j)
                cp = nxt

        pltpu.sync_copy(hstage, out_h.at[pl.ds(base, b_per_w), :])

    return sc_center


def _tc_body(h_ref, csumt_ref, woutt_ref, out_ref, m_ref, s_ref, hb_ref,
             hn_ref, *, B, Wn, V, tile_v, n_pad):
    i = pl.program_id(0)
    nt = pl.num_programs(0)

    @pl.when(i == 0)
    def _init():
        m_ref[...] = jnp.full(m_ref.shape, _NEG, jnp.float32)
        s_ref[...] = jnp.zeros(s_ref.shape, jnp.float32)
        h = h_ref[...]
        hb_ref[...] = (h * _LOG2E).astype(jnp.bfloat16)
        hn_ref[...] = jnp.sqrt(jnp.sum(h * h, axis=1, keepdims=True)) * _LOG2E

    wt = woutt_ref[...]
    col = i * tile_v + lax.broadcasted_iota(jnp.int32, (1, tile_v), 1)
    wt = jnp.where(col < V, wt, 0.0)
    rn2 = jnp.sum(wt * wt, axis=0, keepdims=True)
    mi = jnp.sqrt(jnp.max(rn2, axis=1, keepdims=True))
    b2w = (hn_ref[...] * mi).astype(jnp.bfloat16)
    b2 = b2w.astype(jnp.float32)

    hb = jnp.concatenate([hb_ref[...], b2w], axis=1)
    wb = jnp.concatenate(
        [wt.astype(jnp.bfloat16),
         jnp.full((1, tile_v), -1.0, jnp.bfloat16)], axis=0)
    l2mb = lax.dot_general(hb, wb, (((1,), (0,)), ((), ())),
                           preferred_element_type=jnp.float32)
    t = jnp.sum(jnp.exp2(l2mb), axis=1, keepdims=True)

    m_old = m_ref[...]
    m_new = jnp.maximum(m_old, b2)
    s_ref[...] = (s_ref[...] * jnp.exp2(m_old - m_new)
                  + t * jnp.exp2(b2 - m_new))
    m_ref[...] = m_new

    @pl.when(i == nt - 1)
    def _fin():
        m = m_ref[...]
        s = s_ref[...] - n_pad * jnp.exp2(-m)
        lse = m * _LN2 + jnp.log(s)
        csum = jnp.transpose(csumt_ref[...], (1, 0))
        ctx_total = jnp.sum(h_ref[...] * csum)
        out_ref[0, 0] = jnp.sum(lse) / B - ctx_total / (B * Wn)


@functools.lru_cache(maxsize=None)
def _make_tc_loss(B, Wn, V, D):
    tile_v = _TILE_V
    nt = pl.cdiv(V, tile_v)
    n_pad = nt * tile_v - V
    body = functools.partial(_tc_body, B=B, Wn=Wn, V=V, tile_v=tile_v,
                             n_pad=n_pad)
    return pl.pallas_call(
        body,
        grid=(nt,),
        in_specs=[
            pl.BlockSpec((B, D), lambda i: (0, 0)),
            pl.BlockSpec((D, B), lambda i: (0, 0)),
            pl.BlockSpec((D, tile_v), lambda i: (0, i)),
        ],
        out_specs=pl.BlockSpec((1, 1), lambda i: (0, 0),
                               memory_space=pltpu.SMEM),
        out_shape=jax.ShapeDtypeStruct((1, 1), jnp.float32),
        scratch_shapes=[
            pltpu.VMEM((B, 1), jnp.float32),
            pltpu.VMEM((B, 1), jnp.float32),
            pltpu.VMEM((B, D), jnp.bfloat16),
            pltpu.VMEM((B, 1), jnp.float32),
        ],
    )


def kernel(center_index, context_indices, W_emb, W_out):
    B, Wn = context_indices.shape
    V, D = W_emb.shape
    cidx = center_index.astype(jnp.int32)
    ctx = context_indices.astype(jnp.int32).reshape(-1)
    wembt = jnp.swapaxes(W_emb, 0, 1)
    woutt = jnp.swapaxes(W_out, 0, 1)
    h = _make_sc_center(B, V, D)(cidx, wembt)
    csumt = _make_sc_ctx(B, Wn, V, D)(ctx, woutt)
    loss = _make_tc_loss(B, Wn, V, D)(h, csumt, woutt)
    return loss[0, 0]
